# baseline (speedup 1.0000x reference)
import jax
import jax.numpy as jnp
from jax.experimental import pallas as pl
from jax.experimental.pallas import tpu as pltpu

N = 10000
D_IN = 128
N_H = 128
R = 400
NBLK = N // R


def _dgi_body(seq1_ref, seq2_ref, adj_ref, w_ref, b_ref, sb1_ref, sb2_ref,
              wd_ref, bd_ref, sc1_ref, sc2_ref, h_ref, acc_ref):
    i = pl.program_id(0)

    @pl.when(i == 0)
    def _init():
        acc_ref[:] = jnp.zeros_like(acc_ref)

    a = adj_ref[:]
    a1 = jnp.dot(a, seq1_ref[:], preferred_element_type=jnp.float32)
    a2 = jnp.dot(a, seq2_ref[:], preferred_element_type=jnp.float32)
    h1 = jnp.maximum(
        jnp.dot(a1, w_ref[:], preferred_element_type=jnp.float32) + b_ref[:],
        0.0)
    h2 = jnp.maximum(
        jnp.dot(a2, w_ref[:], preferred_element_type=jnp.float32) + b_ref[:],
        0.0)
    h_ref[pl.ds(i * R, R), :N_H] = h1
    h_ref[pl.ds(i * R, R), N_H:] = h2
    acc_ref[:] += jnp.sum(h1, axis=0, keepdims=True)

    @pl.when(i == NBLK - 1)
    def _finish():
        c = jax.nn.sigmoid(acc_ref[:] / N)
        v = jax.lax.dot_general(c, wd_ref[:], (((1,), (1,)), ((), ())),
                                preferred_element_type=jnp.float32)
        s1 = jax.lax.dot_general(v, h_ref[:, :N_H], (((1,), (1,)), ((), ())),
                                 preferred_element_type=jnp.float32)
        s2 = jax.lax.dot_general(v, h_ref[:, N_H:], (((1,), (1,)), ((), ())),
                                 preferred_element_type=jnp.float32)
        bd = bd_ref[0, 0]
        sc1_ref[:] = s1 + bd + sb1_ref[:]
        sc2_ref[:] = s2 + bd + sb2_ref[:]


def kernel(seq1, seq2, adj, samp_bias1, samp_bias2, W_gcn, b_gcn, W_disc,
           b_disc):
    seq1_2d = seq1.reshape(N, D_IN)
    seq2_2d = seq2.reshape(N, D_IN)
    adj_2d = adj.reshape(N, N)
    b = b_gcn.reshape(1, N_H)
    bd = jnp.asarray(b_disc, jnp.float32).reshape(1, 1)

    full = lambda shape: pl.BlockSpec(shape, lambda i: (0, 0))
    sc1, sc2 = pl.pallas_call(
        _dgi_body,
        grid=(NBLK,),
        in_specs=[
            full((N, D_IN)),
            full((N, D_IN)),
            pl.BlockSpec((R, N), lambda i: (i, 0)),
            full((D_IN, N_H)),
            full((1, N_H)),
            full((1, N)),
            full((1, N)),
            full((N_H, N_H)),
            full((1, 1)),
        ],
        out_specs=[full((1, N)), full((1, N))],
        out_shape=[jax.ShapeDtypeStruct((1, N), jnp.float32),
                   jax.ShapeDtypeStruct((1, N), jnp.float32)],
        scratch_shapes=[
            pltpu.VMEM((N, 2 * N_H), jnp.float32),
            pltpu.VMEM((1, N_H), jnp.float32),
        ],
    )(seq1_2d, seq2_2d, adj_2d, W_gcn, b, samp_bias1, samp_bias2, W_disc, bd)

    return jnp.concatenate([sc1, sc2], axis=1)


# speedup vs baseline: 1.0586x; 1.0586x over previous
import jax
import jax.numpy as jnp
from jax.experimental import pallas as pl
from jax.experimental.pallas import tpu as pltpu

N = 10000
D_IN = 128
N_H = 128
R = 400
NBLK = N // R


def _dgi_body(seq1_ref, seq2_ref, adj_ref, w_ref, b2_ref, sb1_ref, sb2_ref,
              wd_ref, bd_ref, sc1_ref, sc2_ref, x_ref, h_ref, acc_ref):
    i = pl.program_id(0)

    @pl.when(i == 0)
    def _init():
        x_ref[:, :N_H] = jnp.dot(seq1_ref[:], w_ref[:],
                                 preferred_element_type=jnp.float32)
        x_ref[:, N_H:] = jnp.dot(seq2_ref[:], w_ref[:],
                                 preferred_element_type=jnp.float32)
        acc_ref[:] = jnp.zeros_like(acc_ref)

    h = jnp.dot(adj_ref[:], x_ref[:], preferred_element_type=jnp.float32)
    h = jnp.maximum(h + b2_ref[:], 0.0)
    h_ref[pl.ds(i * R, R), :] = h.astype(jnp.bfloat16)
    acc_ref[:] += jnp.sum(h[:, :N_H], axis=0, keepdims=True)

    @pl.when(i == NBLK - 1)
    def _finish():
        c = jax.nn.sigmoid(acc_ref[:] / N)
        v = jax.lax.dot_general(c, wd_ref[:], (((1,), (1,)), ((), ())),
                                preferred_element_type=jnp.float32)
        vb = v.astype(jnp.bfloat16)
        s1 = jax.lax.dot_general(vb, h_ref[:, :N_H], (((1,), (1,)), ((), ())),
                                 preferred_element_type=jnp.float32)
        s2 = jax.lax.dot_general(vb, h_ref[:, N_H:], (((1,), (1,)), ((), ())),
                                 preferred_element_type=jnp.float32)
        bd = bd_ref[0, 0]
        sc1_ref[:] = s1 + bd + sb1_ref[:]
        sc2_ref[:] = s2 + bd + sb2_ref[:]


def kernel(seq1, seq2, adj, samp_bias1, samp_bias2, W_gcn, b_gcn, W_disc,
           b_disc):
    seq1_2d = seq1.reshape(N, D_IN)
    seq2_2d = seq2.reshape(N, D_IN)
    adj_2d = adj.reshape(N, N)
    b2 = jnp.concatenate([b_gcn, b_gcn]).reshape(1, 2 * N_H)
    bd = jnp.asarray(b_disc, jnp.float32).reshape(1, 1)

    full = lambda shape: pl.BlockSpec(shape, lambda i: (0, 0))
    sc1, sc2 = pl.pallas_call(
        _dgi_body,
        grid=(NBLK,),
        in_specs=[
            full((N, D_IN)),
            full((N, D_IN)),
            pl.BlockSpec((R, N), lambda i: (i, 0)),
            full((D_IN, N_H)),
            full((1, 2 * N_H)),
            full((1, N)),
            full((1, N)),
            full((N_H, N_H)),
            full((1, 1)),
        ],
        out_specs=[full((1, N)), full((1, N))],
        out_shape=[jax.ShapeDtypeStruct((1, N), jnp.float32),
                   jax.ShapeDtypeStruct((1, N), jnp.float32)],
        scratch_shapes=[
            pltpu.VMEM((N, 2 * N_H), jnp.float32),
            pltpu.VMEM((N, 2 * N_H), jnp.bfloat16),
            pltpu.VMEM((1, N_H), jnp.float32),
        ],
    )(seq1_2d, seq2_2d, adj_2d, W_gcn, b2, samp_bias1, samp_bias2, W_disc, bd)

    return jnp.concatenate([sc1, sc2], axis=1)
